# baseline (speedup 1.0000x reference)
import jax
import jax.numpy as jnp
from jax.experimental import pallas as pl

N_Q = 8
BINS = 1024
DIM = 256
T = 1024
B = 8


NH = 2
TH = T // NH


def _rvq_body(x_ref, cb_ref, hi_ref, mid_ref, lo_ref,
              out_ref, codes_ref, loss_ref):
    resid = [x_ref[0, :, h * TH:(h + 1) * TH] for h in range(NH)]
    out_acc = [jnp.zeros((DIM, TH), jnp.float32) for _ in range(NH)]
    loss_acc = jnp.float32(0.0)
    code_rows = []
    for j in range(N_Q):
        cb = cb_ref[j]
        norms = jnp.sum(cb * cb, axis=1, keepdims=True)
        idx_halves = []
        for h in range(NH):
            scores = 2.0 * jax.lax.dot_general(
                cb, resid[h],
                dimension_numbers=(((1,), (0,)), ((), ())),
                preferred_element_type=jnp.float32,
            ) - norms
            m = jnp.max(scores, axis=0, keepdims=True)
            bin_ids = jax.lax.broadcasted_iota(jnp.int32, (BINS, TH), 0)
            idx_row = jnp.min(
                jnp.where(scores == m, bin_ids, BINS), axis=0, keepdims=True
            )
            onehot = (bin_ids == idx_row).astype(jnp.bfloat16)
            quant = (jax.lax.dot_general(
                hi_ref[j], onehot,
                dimension_numbers=(((1,), (0,)), ((), ())),
                preferred_element_type=jnp.float32,
            ) + jax.lax.dot_general(
                mid_ref[j], onehot,
                dimension_numbers=(((1,), (0,)), ((), ())),
                preferred_element_type=jnp.float32,
            )) + jax.lax.dot_general(
                lo_ref[j], onehot,
                dimension_numbers=(((1,), (0,)), ((), ())),
                preferred_element_type=jnp.float32,
            )
            diff = quant - resid[h]
            quant_st = resid[h] + diff
            loss_acc = loss_acc + jnp.sum(diff * diff)
            resid[h] = resid[h] - quant_st
            out_acc[h] = out_acc[h] + quant_st
            idx_halves.append(idx_row)
        code_rows.append(jnp.concatenate(idx_halves, axis=1))
    out_ref[0] = jnp.concatenate(out_acc, axis=1)
    codes_ref[0] = jnp.concatenate(code_rows, axis=0)
    b = pl.program_id(0)

    @pl.when(b == 0)
    def _():
        loss_ref[...] = jnp.zeros((1, 1), jnp.float32)

    loss_ref[...] += jnp.reshape(loss_acc * (1.0 / (N_Q * B * DIM * T)), (1, 1))


@jax.jit
def kernel(x, codebooks):
    cbt = jnp.transpose(codebooks, (0, 2, 1))

    def _trunc_split(v):
        bits = jax.lax.bitcast_convert_type(v, jnp.int32)
        top_bf16 = jax.lax.bitcast_convert_type(
            ((bits >> 16) & jnp.int32(0xFFFF)).astype(jnp.uint16), jnp.bfloat16)
        top_f32 = jax.lax.bitcast_convert_type(
            bits & jnp.int32(-65536), jnp.float32)
        return top_bf16, v - top_f32

    hi, r1 = _trunc_split(cbt)
    mid, lo_f = _trunc_split(r1)
    lo = lo_f.astype(jnp.bfloat16)

    out, codes_bt, loss = pl.pallas_call(
        _rvq_body,
        grid=(B,),
        in_specs=[
            pl.BlockSpec((1, DIM, T), lambda b: (b, 0, 0)),
            pl.BlockSpec((N_Q, BINS, DIM), lambda b: (0, 0, 0)),
            pl.BlockSpec((N_Q, DIM, BINS), lambda b: (0, 0, 0)),
            pl.BlockSpec((N_Q, DIM, BINS), lambda b: (0, 0, 0)),
            pl.BlockSpec((N_Q, DIM, BINS), lambda b: (0, 0, 0)),
        ],
        out_specs=[
            pl.BlockSpec((1, DIM, T), lambda b: (b, 0, 0)),
            pl.BlockSpec((1, N_Q, T), lambda b: (b, 0, 0)),
            pl.BlockSpec((1, 1), lambda b: (0, 0)),
        ],
        out_shape=[
            jax.ShapeDtypeStruct((B, DIM, T), jnp.float32),
            jax.ShapeDtypeStruct((B, N_Q, T), jnp.int32),
            jax.ShapeDtypeStruct((1, 1), jnp.float32),
        ],
    )(x, codebooks, hi, mid, lo)
    codes = jnp.transpose(codes_bt, (1, 0, 2))
    return out, codes, loss[0, 0]


# speedup vs baseline: 1.0447x; 1.0447x over previous
import jax
import jax.numpy as jnp
from jax.experimental import pallas as pl

N_Q = 8
BINS = 1024
DIM = 256
T = 1024
B = 8


def _rvq_body(x_ref, cb_ref, hi_ref, mid_ref, lo_ref,
              out_ref, codes_ref, loss_ref):
    resid = x_ref[0]
    loss_mat = jnp.zeros((DIM, T), jnp.float32)
    code_rows = []
    for j in range(N_Q):
        cb = cb_ref[j]
        norms = jnp.sum(cb * cb, axis=1, keepdims=True)
        scores = 2.0 * jax.lax.dot_general(
            cb, resid,
            dimension_numbers=(((1,), (0,)), ((), ())),
            preferred_element_type=jnp.float32,
        ) - norms
        m = jnp.max(scores, axis=0, keepdims=True)
        bin_ids = jax.lax.broadcasted_iota(jnp.int32, (BINS, T), 0)
        idx_row = jnp.min(
            jnp.where(scores == m, bin_ids, BINS), axis=0, keepdims=True
        )
        onehot = (bin_ids == idx_row).astype(jnp.bfloat16)
        quant = (jax.lax.dot_general(
            hi_ref[j], onehot,
            dimension_numbers=(((1,), (0,)), ((), ())),
            preferred_element_type=jnp.float32,
        ) + jax.lax.dot_general(
            mid_ref[j], onehot,
            dimension_numbers=(((1,), (0,)), ((), ())),
            preferred_element_type=jnp.float32,
        )) + jax.lax.dot_general(
            lo_ref[j], onehot,
            dimension_numbers=(((1,), (0,)), ((), ())),
            preferred_element_type=jnp.float32,
        )
        diff = quant - resid
        quant_st = resid + diff
        loss_mat = loss_mat + diff * diff
        resid = resid - quant_st
        code_rows.append(idx_row)
    out_ref[0] = x_ref[0] - resid
    loss_acc = jnp.sum(loss_mat)
    codes_ref[0] = jnp.concatenate(code_rows, axis=0)
    b = pl.program_id(0)

    @pl.when(b == 0)
    def _():
        loss_ref[...] = jnp.zeros((1, 1), jnp.float32)

    loss_ref[...] += jnp.reshape(loss_acc * (1.0 / (N_Q * B * DIM * T)), (1, 1))


@jax.jit
def kernel(x, codebooks):
    cbt = jnp.transpose(codebooks, (0, 2, 1))

    def _trunc_split(v):
        bits = jax.lax.bitcast_convert_type(v, jnp.int32)
        top_bf16 = jax.lax.bitcast_convert_type(
            ((bits >> 16) & jnp.int32(0xFFFF)).astype(jnp.uint16), jnp.bfloat16)
        top_f32 = jax.lax.bitcast_convert_type(
            bits & jnp.int32(-65536), jnp.float32)
        return top_bf16, v - top_f32

    hi, r1 = _trunc_split(cbt)
    mid, lo_f = _trunc_split(r1)
    lo = lo_f.astype(jnp.bfloat16)

    out, codes_bt, loss = pl.pallas_call(
        _rvq_body,
        grid=(B,),
        in_specs=[
            pl.BlockSpec((1, DIM, T), lambda b: (b, 0, 0)),
            pl.BlockSpec((N_Q, BINS, DIM), lambda b: (0, 0, 0)),
            pl.BlockSpec((N_Q, DIM, BINS), lambda b: (0, 0, 0)),
            pl.BlockSpec((N_Q, DIM, BINS), lambda b: (0, 0, 0)),
            pl.BlockSpec((N_Q, DIM, BINS), lambda b: (0, 0, 0)),
        ],
        out_specs=[
            pl.BlockSpec((1, DIM, T), lambda b: (b, 0, 0)),
            pl.BlockSpec((1, N_Q, T), lambda b: (b, 0, 0)),
            pl.BlockSpec((1, 1), lambda b: (0, 0)),
        ],
        out_shape=[
            jax.ShapeDtypeStruct((B, DIM, T), jnp.float32),
            jax.ShapeDtypeStruct((B, N_Q, T), jnp.int32),
            jax.ShapeDtypeStruct((1, 1), jnp.float32),
        ],
    )(x, codebooks, hi, mid, lo)
    codes = jnp.transpose(codes_bt, (1, 0, 2))
    return out, codes, loss[0, 0]


# speedup vs baseline: 1.0464x; 1.0017x over previous
import jax
import jax.numpy as jnp
from jax.experimental import pallas as pl

N_Q = 8
BINS = 1024
DIM = 256
T = 1024
B = 8


def _rvq_body(x_ref, cb_ref, parts_ref, out_ref, codes_ref, loss_ref):
    resid = x_ref[0]
    loss_mat = jnp.zeros((DIM, T), jnp.float32)
    code_rows = []
    for j in range(N_Q):
        cb = cb_ref[j]
        norms = jnp.sum(cb * cb, axis=1, keepdims=True)
        scores = 2.0 * jax.lax.dot_general(
            cb, resid,
            dimension_numbers=(((1,), (0,)), ((), ())),
            preferred_element_type=jnp.float32,
        ) - norms
        m = jnp.max(scores, axis=0, keepdims=True)
        bin_ids = jax.lax.broadcasted_iota(jnp.int32, (BINS, T), 0)
        idx_row = jnp.min(
            jnp.where(scores == m, bin_ids, BINS), axis=0, keepdims=True
        )
        onehot = (bin_ids == idx_row).astype(jnp.bfloat16)
        g = jax.lax.dot_general(
            parts_ref[j], onehot,
            dimension_numbers=(((1,), (0,)), ((), ())),
            preferred_element_type=jnp.float32,
        )
        quant = (g[:DIM] + g[DIM:2 * DIM]) + g[2 * DIM:]
        diff = quant - resid
        quant_st = resid + diff
        loss_mat = loss_mat + diff * diff
        resid = resid - quant_st
        code_rows.append(idx_row)
    out_ref[0] = x_ref[0] - resid
    loss_acc = jnp.sum(loss_mat)
    codes_ref[0] = jnp.concatenate(code_rows, axis=0)
    b = pl.program_id(0)

    @pl.when(b == 0)
    def _():
        loss_ref[...] = jnp.zeros((1, 1), jnp.float32)

    loss_ref[...] += jnp.reshape(loss_acc * (1.0 / (N_Q * B * DIM * T)), (1, 1))


@jax.jit
def kernel(x, codebooks):
    cbt = jnp.transpose(codebooks, (0, 2, 1))

    def _trunc_split(v):
        bits = jax.lax.bitcast_convert_type(v, jnp.int32)
        top_bf16 = jax.lax.bitcast_convert_type(
            ((bits >> 16) & jnp.int32(0xFFFF)).astype(jnp.uint16), jnp.bfloat16)
        top_f32 = jax.lax.bitcast_convert_type(
            bits & jnp.int32(-65536), jnp.float32)
        return top_bf16, v - top_f32

    hi, r1 = _trunc_split(cbt)
    mid, lo_f = _trunc_split(r1)
    lo = lo_f.astype(jnp.bfloat16)
    parts = jnp.concatenate([hi, mid, lo], axis=1)

    out, codes_bt, loss = pl.pallas_call(
        _rvq_body,
        grid=(B,),
        in_specs=[
            pl.BlockSpec((1, DIM, T), lambda b: (b, 0, 0)),
            pl.BlockSpec((N_Q, BINS, DIM), lambda b: (0, 0, 0)),
            pl.BlockSpec((N_Q, 3 * DIM, BINS), lambda b: (0, 0, 0)),
        ],
        out_specs=[
            pl.BlockSpec((1, DIM, T), lambda b: (b, 0, 0)),
            pl.BlockSpec((1, N_Q, T), lambda b: (b, 0, 0)),
            pl.BlockSpec((1, 1), lambda b: (0, 0)),
        ],
        out_shape=[
            jax.ShapeDtypeStruct((B, DIM, T), jnp.float32),
            jax.ShapeDtypeStruct((B, N_Q, T), jnp.int32),
            jax.ShapeDtypeStruct((1, 1), jnp.float32),
        ],
    )(x, codebooks, parts)
    codes = jnp.transpose(codes_bt, (1, 0, 2))
    return out, codes, loss[0, 0]
